# Initial kernel scaffold; baseline (speedup 1.0000x reference)
import jax
import jax.numpy as jnp
from jax.experimental import pallas as pl


def kernel(features, input_mask):
    raise NotImplementedError("write your pallas kernel here")


# speedup vs baseline: 1.6602x; 1.6602x over previous
import jax
import jax.numpy as jnp
from jax.experimental import pallas as pl
from jax.experimental.pallas import tpu as pltpu


def _pool_kernel(x_ref, m_ref, o_ref):
    x = x_ref[...].astype(jnp.float32)
    m = m_ref[...].astype(jnp.float32)
    num = jnp.sum(x * m, axis=1)
    den = jnp.maximum(jnp.sum(m, axis=1), 1.0)
    o_ref[...] = (num * pl.reciprocal(den, approx=False)).astype(o_ref.dtype)


def kernel(features, input_mask):
    B, S, H = features.shape
    itemsize = jnp.dtype(features.dtype).itemsize

    tb = 8
    while (
        tb * 2 <= B
        and B % (tb * 2) == 0
        and 2 * (tb * 2) * S * H * itemsize <= (24 << 20)
        and B // (tb * 2) >= 4
    ):
        tb *= 2
    tb = min(tb, B)

    grid = (pl.cdiv(B, tb),)
    mask3 = input_mask.reshape(B, S, 1)

    feat_block = tb * S * H * itemsize
    mask_block = tb * S * jnp.dtype(input_mask.dtype).itemsize
    out_block = tb * H * itemsize
    vmem_limit = int(min(56 << 20, 2 * (feat_block + mask_block + out_block) + (8 << 20)))

    mask_itemsize = jnp.dtype(input_mask.dtype).itemsize
    cost = pl.CostEstimate(
        flops=2 * B * S * H + B * S + B * H,
        transcendentals=0,
        bytes_accessed=B * S * H * itemsize + B * S * mask_itemsize + B * H * itemsize,
    )

    return pl.pallas_call(
        _pool_kernel,
        out_shape=jax.ShapeDtypeStruct((B, H), features.dtype),
        grid=grid,
        in_specs=[
            pl.BlockSpec((tb, S, H), lambda b: (b, 0, 0)),
            pl.BlockSpec((tb, S, 1), lambda b: (b, 0, 0)),
        ],
        out_specs=pl.BlockSpec((tb, H), lambda b: (b, 0)),
        compiler_params=pltpu.CompilerParams(
            dimension_semantics=("parallel",),
            vmem_limit_bytes=vmem_limit,
        ),
        cost_estimate=cost,
    )(features, mask3)


# speedup vs baseline: 1.6899x; 1.0179x over previous
import jax
import jax.numpy as jnp
from jax.experimental import pallas as pl
from jax.experimental.pallas import tpu as pltpu


def _pool_kernel2(x0_ref, x1_ref, m0_ref, m1_ref, o_ref):
    m0 = m0_ref[...]
    m1 = m1_ref[...]
    num = jnp.sum(x0_ref[...] * m0, axis=1) + jnp.sum(x1_ref[...] * m1, axis=1)
    den = jnp.sum(m0, axis=1) + jnp.sum(m1, axis=1)
    den = jnp.maximum(den, 1.0)
    o_ref[...] = (num * pl.reciprocal(den, approx=False)).astype(o_ref.dtype)


def _pool_kernel1(x_ref, m_ref, o_ref):
    m = m_ref[...]
    num = jnp.sum(x_ref[...] * m, axis=1)
    den = jnp.maximum(jnp.sum(m, axis=1), 1.0)
    o_ref[...] = (num * pl.reciprocal(den, approx=False)).astype(o_ref.dtype)


def kernel(features, input_mask):
    B, S, H = features.shape
    itemsize = jnp.dtype(features.dtype).itemsize
    mask_itemsize = jnp.dtype(input_mask.dtype).itemsize

    tb = 8 if B % 8 == 0 else B
    grid = (pl.cdiv(B, tb),)
    mask3 = input_mask.reshape(B, S, 1)

    feat_block = tb * S * H * itemsize
    mask_block = tb * S * mask_itemsize
    out_block = tb * H * itemsize
    vmem_limit = int(min(56 << 20, 2 * (feat_block + mask_block + out_block) + (12 << 20)))

    cost = pl.CostEstimate(
        flops=2 * B * S * H + B * S + B * H,
        transcendentals=0,
        bytes_accessed=B * S * H * itemsize + B * S * mask_itemsize + B * H * itemsize,
    )
    compiler_params = pltpu.CompilerParams(
        dimension_semantics=("parallel",),
        vmem_limit_bytes=vmem_limit,
    )
    out_shape = jax.ShapeDtypeStruct((B, H), features.dtype)

    if S % 2:
        return pl.pallas_call(
            _pool_kernel1,
            out_shape=out_shape,
            grid=grid,
            in_specs=[
                pl.BlockSpec((tb, S, H), lambda b: (b, 0, 0)),
                pl.BlockSpec((tb, S, 1), lambda b: (b, 0, 0)),
            ],
            out_specs=pl.BlockSpec((tb, H), lambda b: (b, 0)),
            compiler_params=compiler_params,
            cost_estimate=cost,
        )(features, mask3)

    hs = S // 2
    return pl.pallas_call(
        _pool_kernel2,
        out_shape=out_shape,
        grid=grid,
        in_specs=[
            pl.BlockSpec((tb, hs, H), lambda b: (b, 0, 0)),
            pl.BlockSpec((tb, hs, H), lambda b: (b, 1, 0)),
            pl.BlockSpec((tb, hs, 1), lambda b: (b, 0, 0)),
            pl.BlockSpec((tb, hs, 1), lambda b: (b, 1, 0)),
        ],
        out_specs=pl.BlockSpec((tb, H), lambda b: (b, 0)),
        compiler_params=compiler_params,
        cost_estimate=cost,
    )(features, features, mask3, mask3)


# speedup vs baseline: 2.4472x; 1.4481x over previous
import jax
import jax.numpy as jnp
from jax.experimental import pallas as pl
from jax.experimental.pallas import tpu as pltpu


def _pool_kernel(x_ref, m_ref, o_ref):
    x = x_ref[...]
    m = m_ref[...][:, :, None]
    num = jnp.sum(x * m, axis=1)
    den = jnp.maximum(jnp.sum(m, axis=1), 1.0)
    o_ref[...] = (num * pl.reciprocal(den, approx=False)).astype(o_ref.dtype)


def kernel(features, input_mask):
    B, S, H = features.shape
    itemsize = jnp.dtype(features.dtype).itemsize
    mask_itemsize = jnp.dtype(input_mask.dtype).itemsize

    tb = 8 if B % 8 == 0 else B
    grid = (pl.cdiv(B, tb),)

    feat_block = tb * S * H * itemsize
    mask_block = tb * S * mask_itemsize
    out_block = tb * H * itemsize
    vmem_limit = int(min(56 << 20, 2 * (feat_block + mask_block + out_block) + (12 << 20)))

    cost = pl.CostEstimate(
        flops=2 * B * S * H + B * S + B * H,
        transcendentals=0,
        bytes_accessed=B * S * H * itemsize + B * S * mask_itemsize + B * H * itemsize,
    )

    return pl.pallas_call(
        _pool_kernel,
        out_shape=jax.ShapeDtypeStruct((B, H), features.dtype),
        grid=grid,
        in_specs=[
            pl.BlockSpec((tb, S, H), lambda b: (b, 0, 0)),
            pl.BlockSpec((tb, S), lambda b: (b, 0)),
        ],
        out_specs=pl.BlockSpec((tb, H), lambda b: (b, 0)),
        compiler_params=pltpu.CompilerParams(
            dimension_semantics=("parallel",),
            vmem_limit_bytes=vmem_limit,
        ),
        cost_estimate=cost,
    )(features, input_mask)
